# Initial kernel scaffold; baseline (speedup 1.0000x reference)
import jax
import jax.numpy as jnp
from jax.experimental import pallas as pl


def kernel(x, W):
    raise NotImplementedError("write your pallas kernel here")


# speedup vs baseline: 1.0900x; 1.0900x over previous
import functools

import jax
import jax.numpy as jnp
from jax import lax
from jax.experimental import pallas as pl
from jax.experimental.pallas import tpu as pltpu
from jax.experimental.pallas import tpu_sc as plsc

NC = 2
NS = 16
NW = NC * NS
IDX_W = 128


@functools.lru_cache(maxsize=None)
def _make_gather(V, D, B, chunk_rows):
    rows_per_w = B // NW
    ivecs_per_w = rows_per_w // IDX_W
    ivecs_per_chunk = chunk_rows // IDX_W
    n_chunks = rows_per_w // chunk_rows
    mesh = plsc.VectorSubcoreMesh(core_axis_name="c", subcore_axis_name="s")

    @functools.partial(
        pl.kernel,
        out_type=jax.ShapeDtypeStruct((B, D), jnp.float32),
        mesh=mesh,
        scratch_types=[
            pltpu.VMEM((ivecs_per_w, IDX_W), jnp.int32),
            pltpu.VMEM((chunk_rows, D), jnp.float32),
            pltpu.SemaphoreType.DMA,
        ],
        compiler_params=pltpu.CompilerParams(use_tc_tiling_on_sc=False),
    )
    def k(table_hbm, idx_hbm, out_hbm, idx_v, rows_v, sem):
        wid = lax.axis_index("s") * NC + lax.axis_index("c")
        pltpu.sync_copy(idx_hbm.at[pl.ds(wid * ivecs_per_w, ivecs_per_w)], idx_v)
        out_base = wid * rows_per_w

        def chunk_body(c, carry):
            copies = []
            for j in range(ivecs_per_chunk):
                copies.append(pltpu.async_copy(
                    table_hbm.at[idx_v.at[c * ivecs_per_chunk + j]],
                    rows_v.at[pl.ds(j * IDX_W, IDX_W)],
                    sem,
                ))
            for cp in copies:
                cp.wait()
            pltpu.sync_copy(rows_v, out_hbm.at[pl.ds(out_base + c * chunk_rows,
                                                     chunk_rows)])
            return carry

        lax.fori_loop(0, n_chunks, chunk_body, 0)

    return k


def kernel(x, W):
    V, D = W.shape
    B = x.size
    idx = x.reshape(B // IDX_W, IDX_W).astype(jnp.int32)
    out = _make_gather(V, D, B, 512)(W, idx)
    return out.reshape(x.shape + (D,))


# speedup vs baseline: 1.1080x; 1.0165x over previous
import functools

import jax
import jax.numpy as jnp
from jax import lax
from jax.experimental import pallas as pl
from jax.experimental.pallas import tpu as pltpu
from jax.experimental.pallas import tpu_sc as plsc

NC = 2
NS = 16
NW = NC * NS
IDX_W = 128


@functools.lru_cache(maxsize=None)
def _make_gather(V, D, B, chunk_rows):
    rows_per_w = B // NW
    ivecs_per_w = rows_per_w // IDX_W
    ivecs_per_chunk = chunk_rows // IDX_W
    n_chunks = rows_per_w // chunk_rows
    mesh = plsc.VectorSubcoreMesh(core_axis_name="c", subcore_axis_name="s")

    assert n_chunks % 2 == 0 and n_chunks >= 4

    @functools.partial(
        pl.kernel,
        out_type=jax.ShapeDtypeStruct((B, D), jnp.float32),
        mesh=mesh,
        scratch_types=[
            pltpu.VMEM((ivecs_per_w, IDX_W), jnp.int32),
            pltpu.VMEM((2, chunk_rows, D), jnp.float32),
            pltpu.SemaphoreType.DMA,
            pltpu.SemaphoreType.DMA,
            pltpu.SemaphoreType.DMA,
            pltpu.SemaphoreType.DMA,
        ],
        compiler_params=pltpu.CompilerParams(use_tc_tiling_on_sc=False),
    )
    def k(table_hbm, idx_hbm, out_hbm, idx_v, rows_v, si0, si1, so0, so1):
        wid = lax.axis_index("s") * NC + lax.axis_index("c")
        sems_in = (si0, si1)
        sems_out = (so0, so1)
        pltpu.sync_copy(idx_hbm.at[pl.ds(wid * ivecs_per_w, ivecs_per_w)], idx_v)
        out_base = wid * rows_per_w

        def issue_gathers(c, b):
            copies = []
            for j in range(ivecs_per_chunk):
                copies.append(pltpu.async_copy(
                    table_hbm.at[idx_v.at[c * ivecs_per_chunk + j]],
                    rows_v.at[b].at[pl.ds(j * IDX_W, IDX_W)],
                    sems_in[b],
                ))
            return copies

        def issue_out(c, b):
            return pltpu.async_copy(
                rows_v.at[b],
                out_hbm.at[pl.ds(out_base + c * chunk_rows, chunk_rows)],
                sems_out[b],
            )

        def drain_out(c, b):
            pltpu.make_async_copy(
                rows_v.at[b],
                out_hbm.at[pl.ds(out_base + c * chunk_rows, chunk_rows)],
                sems_out[b],
            ).wait()

        g0 = issue_gathers(0, 0)
        g1 = issue_gathers(1, 1)
        for cp in g0:
            cp.wait()
        issue_out(0, 0)
        for cp in g1:
            cp.wait()
        issue_out(1, 1)

        def pair_body(g, carry):
            c = 2 * g
            drain_out(c, 0)
            g0 = issue_gathers(c, 0)
            drain_out(c + 1, 1)
            g1 = issue_gathers(c + 1, 1)
            for cp in g0:
                cp.wait()
            issue_out(c, 0)
            for cp in g1:
                cp.wait()
            issue_out(c + 1, 1)
            return carry

        lax.fori_loop(1, n_chunks // 2, pair_body, 0)

        drain_out(n_chunks - 2, 0)
        drain_out(n_chunks - 1, 1)

    return k


def kernel(x, W):
    V, D = W.shape
    B = x.size
    idx = x.reshape(B // IDX_W, IDX_W).astype(jnp.int32)
    out = _make_gather(V, D, B, 512)(W, idx)
    return out.reshape(x.shape + (D,))
